# Initial kernel scaffold; baseline (speedup 1.0000x reference)
import jax
import jax.numpy as jnp
from jax.experimental import pallas as pl


def kernel(entity_embd, edge_index, edge_attr, q_embd, Ws0, bs0, Wn0, bn0, Ws1, bs1, Wn1, bn1, Wr1, br1, Wr2, br2, Wp1, bp1, Wp2, bp2):
    raise NotImplementedError("write your pallas kernel here")


# speedup vs baseline: 5.1897x; 5.1897x over previous
import functools

import jax
import jax.numpy as jnp
from jax import lax
from jax.experimental import pallas as pl
from jax.experimental.pallas import tpu as pltpu
from jax.experimental.pallas import tpu_sc as plsc

N = 10000
E = 320000
D = 128
K = 2048

NC = 2
NS = 16
NW = NC * NS
EW = E // NW
SC_B = 80
SC_BG = 400


def _row_split_copy(s, mk_src, mk_dst):
    @pl.when(s < 2)
    def _():
        off = pl.multiple_of(s * 632, 8)
        pltpu.sync_copy(mk_src(off, 632), mk_dst(off, 632))

    @pl.when(s >= 2)
    def _():
        off = pl.multiple_of(1264 + (s - 2) * 624, 8)
        pltpu.sync_copy(mk_src(off, 624), mk_dst(off, 624))

_sc_mesh = functools.partial(
    plsc.VectorSubcoreMesh, core_axis_name="c", subcore_axis_name="s")


def _edge_mlp_body(ea, wr1, br1, wr2, br2, out):
    h = jnp.maximum(ea[...] @ wr1[...] + br1[...], 0.0)
    out[...] = h @ wr2[...] + br2[...]


def _edge_mlp(ea, Wr1, br1, Wr2, br2):
    blk = 2000
    grid = E // blk
    return pl.pallas_call(
        _edge_mlp_body,
        grid=(grid,),
        in_specs=[
            pl.BlockSpec((blk, D), lambda i: (i, 0)),
            pl.BlockSpec((D, D), lambda i: (0, 0)),
            pl.BlockSpec((1, D), lambda i: (0, 0)),
            pl.BlockSpec((D, D), lambda i: (0, 0)),
            pl.BlockSpec((1, D), lambda i: (0, 0)),
        ],
        out_specs=pl.BlockSpec((blk, D), lambda i: (i, 0)),
        out_shape=jax.ShapeDtypeStruct((E, D), jnp.float32),
    )(ea, Wr1, br1.reshape(1, D), Wr2, br2.reshape(1, D))


def _shr_body(ea_h, ear_h, hid_h, tid_h, z128_h, z1_h,
              s_out, deg_out, hidx_v, tidx_v, rows_v, ones_v, s_acc, deg_acc):
    c = lax.axis_index("c")
    s = lax.axis_index("s")
    wid = s * NC + c
    _row_split_copy(s, lambda o, n: z128_h.at[pl.ds(o, n)],
                    lambda o, n: s_acc.at[pl.ds(o, n)])

    @pl.when(s == 0)
    def _():
        pltpu.sync_copy(z1_h, deg_acc)

    for j in range(SC_B // 16):
        ones_v[pl.ds(j * 16, 16)] = jnp.ones((16,), jnp.float32)
    plsc.subcore_barrier()

    base0 = wid * EW

    def batch(i, carry):
        base = base0 + i * SC_B
        pltpu.sync_copy(hid_h.at[pl.ds(base, SC_B)], hidx_v)
        pltpu.sync_copy(tid_h.at[pl.ds(base, SC_B)], tidx_v)
        pltpu.sync_copy(ea_h.at[pl.ds(base, SC_B)], rows_v)
        pltpu.sync_copy(rows_v, s_acc.at[tidx_v], add=True)
        pltpu.sync_copy(ones_v, deg_acc.at[tidx_v], add=True)
        pltpu.sync_copy(ear_h.at[pl.ds(base, SC_B)], rows_v)
        pltpu.sync_copy(rows_v, s_acc.at[hidx_v], add=True)
        pltpu.sync_copy(ones_v, deg_acc.at[hidx_v], add=True)
        return carry

    lax.fori_loop(0, EW // SC_B, batch, 0)
    plsc.subcore_barrier()
    _row_split_copy(s, lambda o, n: s_acc.at[pl.ds(o, n)],
                    lambda o, n: s_out.at[c, pl.ds(o, n)])

    @pl.when(s == 0)
    def _():
        pltpu.sync_copy(deg_acc, deg_out.at[c])


def _shr_scatter(ea, ear, h_id, t_id, z128, z1):
    f = pl.kernel(
        _shr_body,
        out_type=[jax.ShapeDtypeStruct((NC, N, D), jnp.float32),
                  jax.ShapeDtypeStruct((NC, N), jnp.float32)],
        mesh=_sc_mesh(),
        scratch_types=[
            pltpu.VMEM((SC_B,), jnp.int32),
            pltpu.VMEM((SC_B,), jnp.int32),
            pltpu.VMEM((SC_B, D), jnp.float32),
            pltpu.VMEM((SC_B,), jnp.float32),
            pltpu.VMEM_SHARED((N, D), jnp.float32),
            pltpu.VMEM_SHARED((N,), jnp.float32),
        ],
    )
    return f(ea, ear, h_id, t_id, z128, z1)


def _agg_body(x_h, hid_h, tid_h, z128_h,
              out, hidx_v, tidx_v, rows_v, acc):
    c = lax.axis_index("c")
    s = lax.axis_index("s")
    wid = s * NC + c
    _row_split_copy(s, lambda o, n: z128_h.at[pl.ds(o, n)],
                    lambda o, n: acc.at[pl.ds(o, n)])
    plsc.subcore_barrier()

    base0 = wid * EW

    def batch(i, carry):
        base = base0 + i * SC_B
        pltpu.sync_copy(hid_h.at[pl.ds(base, SC_B)], hidx_v)
        pltpu.sync_copy(tid_h.at[pl.ds(base, SC_B)], tidx_v)
        pltpu.sync_copy(x_h.at[hidx_v], rows_v)
        pltpu.sync_copy(rows_v, acc.at[tidx_v], add=True)
        pltpu.sync_copy(x_h.at[tidx_v], rows_v)
        pltpu.sync_copy(rows_v, acc.at[hidx_v], add=True)
        return carry

    lax.fori_loop(0, EW // SC_B, batch, 0)
    plsc.subcore_barrier()
    _row_split_copy(s, lambda o, n: acc.at[pl.ds(o, n)],
                    lambda o, n: out.at[c, pl.ds(o, n)])


def _agg_scatter(x, h_id, t_id, z128):
    f = pl.kernel(
        _agg_body,
        out_type=jax.ShapeDtypeStruct((NC, N, D), jnp.float32),
        mesh=_sc_mesh(),
        scratch_types=[
            pltpu.VMEM((SC_B,), jnp.int32),
            pltpu.VMEM((SC_B,), jnp.int32),
            pltpu.VMEM((SC_B, D), jnp.float32),
            pltpu.VMEM_SHARED((N, D), jnp.float32),
        ],
    )
    return f(x, h_id, t_id, z128)


def _gath_body(a_h, b_h, hid_h, tid_h, g1_out, g2_out,
               hidx_v, tidx_v, rows_v):
    c = lax.axis_index("c")
    s = lax.axis_index("s")
    wid = s * NC + c
    base0 = wid * EW

    def batch(i, carry):
        base = base0 + i * SC_BG
        pltpu.sync_copy(hid_h.at[pl.ds(base, SC_BG)], hidx_v)
        pltpu.sync_copy(tid_h.at[pl.ds(base, SC_BG)], tidx_v)
        pltpu.sync_copy(a_h.at[hidx_v], rows_v)
        pltpu.sync_copy(rows_v, g1_out.at[pl.ds(base, SC_BG)])
        pltpu.sync_copy(b_h.at[tidx_v], rows_v)
        pltpu.sync_copy(rows_v, g2_out.at[pl.ds(base, SC_BG)])
        return carry

    lax.fori_loop(0, EW // SC_BG, batch, 0)


def _edge_gathers(A, Bm, h_id, t_id):
    f = pl.kernel(
        _gath_body,
        out_type=[jax.ShapeDtypeStruct((E, D), jnp.float32),
                  jax.ShapeDtypeStruct((E, D), jnp.float32)],
        mesh=_sc_mesh(),
        scratch_types=[
            pltpu.VMEM((SC_BG,), jnp.int32),
            pltpu.VMEM((SC_BG,), jnp.int32),
            pltpu.VMEM((SC_BG, D), jnp.float32),
        ],
    )
    return f(A, Bm, h_id, t_id)


def _layer_body(x, s0, s1, a0, a1, d0, d1, ws, bs, wn, bn, out):
    deg = jnp.maximum(d0[...] + d1[...], 1.0)
    agg = (a0[...] + a1[...] + s0[...] + s1[...]) / deg
    out[...] = jnp.maximum(
        x[...] @ ws[...] + bs[...] + agg @ wn[...] + bn[...], 0.0)


def _layer(x, S0, S1, A0, A1, d0, d1, Ws, bs, Wn, bn):
    blk = 2000
    grid = N // blk
    row = lambda i: (i, 0)
    full = lambda i: (0, 0)
    return pl.pallas_call(
        _layer_body,
        grid=(grid,),
        in_specs=[pl.BlockSpec((blk, D), row)] * 5
        + [pl.BlockSpec((blk, 1), row)] * 2
        + [pl.BlockSpec((D, D), full), pl.BlockSpec((1, D), full)] * 2,
        out_specs=pl.BlockSpec((blk, D), row),
        out_shape=jax.ShapeDtypeStruct((N, D), jnp.float32),
    )(x, S0, S1, A0, A1, d0, d1, Ws, bs.reshape(1, D), Wn, bn.reshape(1, D))


def _layer2_body(x, s0, s1, a0, a1, d0, d1, ws, bs, wn, bn,
                 wa1, wa2, wb1, wb2, a_out, b_out):
    deg = jnp.maximum(d0[...] + d1[...], 1.0)
    agg = (a0[...] + a1[...] + s0[...] + s1[...]) / deg
    x2 = jnp.maximum(
        x[...] @ ws[...] + bs[...] + agg @ wn[...] + bn[...], 0.0)
    a_out[...] = x[...] @ wa1[...] + x2 @ wa2[...]
    b_out[...] = x[...] @ wb1[...] + x2 @ wb2[...]


def _layer2(x, S0, S1, A0, A1, d0, d1, Ws, bs, Wn, bn, Wa1, Wa2, Wb1, Wb2):
    blk = 2000
    grid = N // blk
    row = lambda i: (i, 0)
    full = lambda i: (0, 0)
    return pl.pallas_call(
        _layer2_body,
        grid=(grid,),
        in_specs=[pl.BlockSpec((blk, D), row)] * 5
        + [pl.BlockSpec((blk, 1), row)] * 2
        + [pl.BlockSpec((D, D), full), pl.BlockSpec((1, D), full)] * 2
        + [pl.BlockSpec((D, D), full)] * 4,
        out_specs=[pl.BlockSpec((blk, D), row)] * 2,
        out_shape=[jax.ShapeDtypeStruct((N, D), jnp.float32),
                   jax.ShapeDtypeStruct((N, D), jnp.float32)],
    )(x, S0, S1, A0, A1, d0, d1, Ws, bs.reshape(1, D), Wn, bn.reshape(1, D),
      Wa1, Wa2, Wb1, Wb2)


def _score_body(ea, g1, g2, q, wq, bp1, we, wp2, bp2, out):
    qc = q[...] @ wq[...] + bp1[...]
    hidden = jnp.maximum(qc + g1[...] + g2[...] + ea[...] @ we[...], 0.0)
    out[...] = hidden @ wp2[...] + bp2[...]


def _score(ea, G1, G2, q, Wq, bp1, We, Wp2, bp2):
    blk = 2000
    grid = E // blk
    row = lambda i: (i, 0)
    full = lambda i: (0, 0)
    return pl.pallas_call(
        _score_body,
        grid=(grid,),
        in_specs=[pl.BlockSpec((blk, D), row)] * 3
        + [pl.BlockSpec((1, D), full), pl.BlockSpec((D, D), full),
           pl.BlockSpec((1, D), full), pl.BlockSpec((D, D), full),
           pl.BlockSpec((D, 1), full), pl.BlockSpec((1, 1), full)],
        out_specs=pl.BlockSpec((blk, 1), row),
        out_shape=jax.ShapeDtypeStruct((E, 1), jnp.float32),
    )(ea, G1, G2, q, Wq, bp1.reshape(1, D), We, Wp2, bp2.reshape(1, 1))


_RY = E // D


def _avg_floor(a, b):
    return (a >> 1) + (b >> 1) + (a & b & 1)


def _topk_body(l_ref, g_ref, out):
    y = l_ref[...] + g_ref[...]
    bits = lax.bitcast_convert_type(y, jnp.int32)
    imin = jnp.int32(-2147483648)
    key = jnp.where(bits >= 0, bits, imin - bits)

    kk = jnp.int32(K)

    def bs1(i, c):
        lo, hi = c
        mid = _avg_floor(lo, hi) + ((lo ^ hi) & 1)
        cnt = jnp.sum((key >= mid).astype(jnp.int32))
        ok = cnt >= kk
        return (jnp.where(ok, mid, lo), jnp.where(ok, hi, mid - 1))

    lo, hi = lax.fori_loop(0, 33, bs1, (jnp.min(key), jnp.max(key)))
    t = lo

    cnt_gt = jnp.sum((key > t).astype(jnp.int32))
    need = kk - cnt_gt
    tie = key == t
    r = lax.broadcasted_iota(jnp.int32, (_RY, D), 0)
    col = lax.broadcasted_iota(jnp.int32, (_RY, D), 1)
    pos = r * D + col

    def bs2(i, c):
        lo2, hi2 = c
        mid = (lo2 + hi2) >> 1
        cnt = jnp.sum((tie & (pos < mid)).astype(jnp.int32))
        ok = cnt >= need
        return (jnp.where(ok, lo2, mid + 1), jnp.where(ok, mid, hi2))

    lo2, hi2 = lax.fori_loop(0, 20, bs2, (jnp.int32(0), jnp.int32(E)))
    sel = (key > t) | (tie & (pos < hi2))
    out[...] = sel.astype(jnp.float32)


def _topk_mask(l2d, g2d):
    return pl.pallas_call(
        _topk_body,
        out_shape=jax.ShapeDtypeStruct((_RY, D), jnp.float32),
    )(l2d, g2d)


def kernel(entity_embd, edge_index, edge_attr, q_embd,
           Ws0, bs0, Wn0, bn0, Ws1, bs1, Wn1, bn1,
           Wr1, br1, Wr2, br2, Wp1, bp1, Wp2, bp2):
    h_id = edge_index[0]
    t_id = edge_index[1]
    z128 = jnp.zeros((N, D), jnp.float32)
    z1 = jnp.zeros((N,), jnp.float32)

    ear = _edge_mlp(edge_attr, Wr1, br1, Wr2, br2)
    S_parts, deg_parts = _shr_scatter(edge_attr, ear, h_id, t_id, z128, z1)
    S0, S1 = S_parts[0], S_parts[1]
    d0 = deg_parts[0].reshape(N, 1)
    d1 = deg_parts[1].reshape(N, 1)

    x0, _ = lax.optimization_barrier((entity_embd, deg_parts))
    agg0 = _agg_scatter(x0, h_id, t_id, z128)
    x1 = _layer(entity_embd, S0, S1, agg0[0], agg0[1], d0, d1,
                Ws0, bs0, Wn0, bn0)
    agg1 = _agg_scatter(x1, h_id, t_id, z128)
    A, Bm = _layer2(x1, S0, S1, agg1[0], agg1[1], d0, d1,
                    Ws1, bs1, Wn1, bn1,
                    Wp1[128:256], Wp1[256:384], Wp1[512:640], Wp1[640:768])

    G1, G2 = _edge_gathers(A, Bm, h_id, t_id)
    logits = _score(edge_attr, G1, G2, q_embd, Wp1[0:128], bp1,
                    Wp1[384:512], Wp2, bp2)

    gkey = jax.random.key(42)
    u = jax.random.uniform(gkey, (E, 1), minval=1e-10, maxval=1.0 - 1e-10)
    g = -jnp.log(-jnp.log(u))
    mask2d = _topk_mask(logits.reshape(_RY, D), g.reshape(_RY, D))
    return logits, mask2d.reshape(E, 1)
